# baseline (speedup 1.0000x reference)
import functools

import jax
import jax.numpy as jnp
from jax import lax
from jax.experimental import pallas as pl
from jax.experimental.pallas import tpu as pltpu
from jax.experimental.pallas import tpu_sc as plsc

B, C, K = 4, 96, 1024
H = W = 384
NC, NS, L = 2, 16, 16
CH = 4
CB = C // CH
PB = (NC * NS) // CH
BH, BW = 8, 128
WB = W // BW
HB_PER_W = (H // BH) // PB
NBLK = HB_PER_W * WB
TOT = B * NBLK
NBUF = 3
GRP = BW // L


def _unpool_sc(pooled_flat, spx):
    mesh = plsc.VectorSubcoreMesh(core_axis_name="c", subcore_axis_name="s")

    @functools.partial(
        pl.kernel,
        mesh=mesh,
        out_type=jax.ShapeDtypeStruct((B, C, H, W), jnp.float32),
        compiler_params=pltpu.CompilerParams(needs_layout_passes=False),
        scratch_types=[
            pltpu.VMEM((CB * K,), jnp.float32),
            pltpu.VMEM((NBUF, BH, BW), jnp.int32),
            pltpu.VMEM((NBUF, CB, BH, BW), jnp.float32),
            pltpu.SemaphoreType.DMA((NBUF,)),
            pltpu.SemaphoreType.DMA((NBUF,)),
        ],
    )
    def unpool(pooled_hbm, spx_hbm, out_hbm, table_v, idx_v, out_v, isem, osem):
        wid = lax.axis_index("s") * NC + lax.axis_index("c")
        ch = wid % CH
        pb = wid // CH
        c0 = ch * CB

        def start_idx(b, hoff, wb, p):
            h0 = (pb * HB_PER_W + hoff) * BH
            pltpu.make_async_copy(
                spx_hbm.at[b, pl.ds(h0, BH), pl.ds(wb * BW, BW)],
                idx_v.at[p],
                isem.at[p],
            ).start()

        def wait_idx(p):
            pltpu.make_async_copy(
                spx_hbm.at[0, pl.ds(0, BH), pl.ds(0, BW)],
                idx_v.at[p],
                isem.at[p],
            ).wait()

        def start_out(b, hoff, wb, p):
            h0 = (pb * HB_PER_W + hoff) * BH
            pltpu.make_async_copy(
                out_v.at[p],
                out_hbm.at[
                    b, pl.ds(c0, CB), pl.ds(h0, BH), pl.ds(wb * BW, BW)
                ],
                osem.at[p],
            ).start()

        def wait_out(p):
            pltpu.make_async_copy(
                out_v.at[p],
                out_hbm.at[0, pl.ds(0, CB), pl.ds(0, BH), pl.ds(0, BW)],
                osem.at[p],
            ).wait()

        start_idx(0, 0, 0, 0)

        def step(i, carry):
            b, t, hoff, wb, p = carry

            @pl.when(t == 0)
            def _():
                pltpu.sync_copy(
                    pooled_hbm.at[b, pl.ds(c0 * K, CB * K)], table_v
                )

            last_t = t == NBLK - 1
            last_w = wb == WB - 1
            nt = jnp.where(last_t, 0, t + 1)
            nb = jnp.where(last_t, b + 1, b)
            nwb = jnp.where(last_w, 0, wb + 1)
            nhoff = jnp.where(last_t, 0, jnp.where(last_w, hoff + 1, hoff))
            np_ = jnp.where(p == NBUF - 1, 0, p + 1)

            @pl.when(i + 1 < TOT)
            def _():
                start_idx(nb, nhoff, nwb, np_)

            wait_idx(p)

            @pl.when(i >= NBUF)
            def _():
                wait_out(p)

            for r in range(BH):
                for g in range(GRP):
                    pix = idx_v[p, r, pl.ds(g * L, L)]
                    for c in range(0, CB, 8):
                        vals = [
                            plsc.load_gather(table_v, [pix + (c + j) * K])
                            for j in range(8)
                        ]
                        for j in range(8):
                            out_v[p, c + j, r, pl.ds(g * L, L)] = vals[j]

            start_out(b, hoff, wb, p)
            return (nb, nt, nhoff, nwb, np_)

        zero = jnp.int32(0)
        lax.fori_loop(0, TOT, step, (zero, zero, zero, zero, zero),
                      unroll=False)
        for i in range(NBUF):
            wait_out((TOT - NBUF + i) % NBUF)

    return unpool(pooled_flat, spx)


def kernel(pooled, spx):
    pooled_flat = pooled.reshape(B, C * K)
    return _unpool_sc(pooled_flat, spx)


# speedup vs baseline: 2.1417x; 2.1417x over previous
import functools

import jax
import jax.numpy as jnp
from jax import lax
from jax.experimental import pallas as pl
from jax.experimental.pallas import tpu as pltpu
from jax.experimental.pallas import tpu_sc as plsc

B, C, K = 4, 96, 1024
H = W = 384
NC, NS, L = 2, 16, 16
NW = NC * NS
CB = C // NW
BH = 32
NBLK = H // BH
TOT = B * NBLK
NBUF = 2
GRP = W // L


def _unpool_sc(pooled_flat, spx):
    mesh = plsc.VectorSubcoreMesh(core_axis_name="c", subcore_axis_name="s")

    @functools.partial(
        pl.kernel,
        mesh=mesh,
        out_type=jax.ShapeDtypeStruct((B, C, H, W), jnp.float32),
        compiler_params=pltpu.CompilerParams(needs_layout_passes=False),
        scratch_types=[
            pltpu.VMEM((CB * K,), jnp.float32),
            pltpu.VMEM((NBUF, BH, W), jnp.int32),
            pltpu.VMEM((NBUF, CB, BH, W), jnp.float32),
            pltpu.SemaphoreType.DMA((NBUF,)),
            pltpu.SemaphoreType.DMA((NBUF,)),
        ],
    )
    def unpool(pooled_hbm, spx_hbm, out_hbm, table_v, idx_v, out_v, isem, osem):
        wid = lax.axis_index("s") * NC + lax.axis_index("c")
        c0 = wid * CB

        def start_idx(b, t, p):
            pltpu.make_async_copy(
                spx_hbm.at[b, pl.ds(t * BH, BH), :],
                idx_v.at[p],
                isem.at[p],
            ).start()

        def wait_idx(p):
            pltpu.make_async_copy(
                spx_hbm.at[0, pl.ds(0, BH), :],
                idx_v.at[p],
                isem.at[p],
            ).wait()

        def start_out(b, t, p):
            pltpu.make_async_copy(
                out_v.at[p],
                out_hbm.at[b, pl.ds(c0, CB), pl.ds(t * BH, BH), :],
                osem.at[p],
            ).start()

        def wait_out(p):
            pltpu.make_async_copy(
                out_v.at[p],
                out_hbm.at[0, pl.ds(0, CB), pl.ds(0, BH), :],
                osem.at[p],
            ).wait()

        start_idx(0, 0, 0)

        def step(i, carry):
            b, t, p = carry

            @pl.when(t == 0)
            def _():
                pltpu.sync_copy(
                    pooled_hbm.at[b, pl.ds(c0 * K, CB * K)], table_v
                )

            last_t = t == NBLK - 1
            nt = jnp.where(last_t, 0, t + 1)
            nb = jnp.where(last_t, b + 1, b)
            np_ = 1 - p

            @pl.when(i + 1 < TOT)
            def _():
                start_idx(nb, nt, np_)

            wait_idx(p)

            @pl.when(i >= NBUF)
            def _():
                wait_out(p)

            @plsc.parallel_loop(0, BH, unroll=1)
            def row_body(r):
                for g in range(GRP):
                    pix = idx_v[p, r, pl.ds(g * L, L)]
                    vals = [
                        plsc.load_gather(table_v, [pix + c * K])
                        for c in range(CB)
                    ]
                    for c in range(CB):
                        out_v[p, c, r, pl.ds(g * L, L)] = vals[c]

            start_out(b, t, p)
            return (nb, nt, np_)

        zero = jnp.int32(0)
        lax.fori_loop(0, TOT, step, (zero, zero, zero), unroll=False)
        for i in range(NBUF):
            wait_out((TOT - NBUF + i) % NBUF)

    return unpool(pooled_flat, spx)


def kernel(pooled, spx):
    pooled_flat = pooled.reshape(B, C * K)
    return _unpool_sc(pooled_flat, spx)
